# Initial kernel scaffold; baseline (speedup 1.0000x reference)
import functools

import jax
import jax.numpy as jnp
from jax.experimental import pallas as pl
from jax.experimental.pallas import tpu as pltpu

_B, _N, _D, _H = 32, 577, 768, 12
_DH = _D // _H
_P = _N - 1
_K = 144
_TAU = 2.0


def _ln(v, w, b, eps=1e-5):
    mu = v.mean(axis=-1, keepdims=True)
    var = ((v - mu) ** 2).mean(axis=-1, keepdims=True)
    return (v - mu) / jnp.sqrt(var + eps) * w + b


def _attn_body(x_ref, n1w, n1b, qkvw, qkvb, pw, pb, ls1, x1_ref, ca_ref):
    xb = x_ref[0]
    h = _ln(xb, n1w[:], n1b[:])
    qkv = jnp.dot(h, qkvw[:], preferred_element_type=jnp.float32) + qkvb[:]
    acc = jnp.zeros((_N, _D), jnp.float32)
    ca = jnp.zeros((1, _N), jnp.float32)
    scale = _DH ** -0.5
    for hh in range(_H):
        q = qkv[:, hh * _DH:(hh + 1) * _DH]
        k = qkv[:, _D + hh * _DH:_D + (hh + 1) * _DH]
        v = qkv[:, 2 * _D + hh * _DH:2 * _D + (hh + 1) * _DH]
        s = jax.lax.dot_general(q, k, (((1,), (1,)), ((), ())),
                                preferred_element_type=jnp.float32) * scale
        m = jnp.max(s, axis=-1, keepdims=True)
        e = jnp.exp(s - m)
        p = e / jnp.sum(e, axis=-1, keepdims=True)
        o = jnp.dot(p, v, preferred_element_type=jnp.float32)
        acc = acc + jnp.dot(o, pw[hh * _DH:(hh + 1) * _DH, :],
                            preferred_element_type=jnp.float32)
        ca = ca + p[0:1, :]
    x1_ref[0] = xb + ls1[:] * (acc + pb[:])
    ca_ref[0] = ca / 12.0


def _mlp_body(x1_ref, n2w, n2b, f1w, f1b, f2w, f2b, ls2, x2_ref):
    xb = x1_ref[0]
    h2 = _ln(xb, n2w[:], n2b[:])
    a = jnp.dot(h2, f1w[:], preferred_element_type=jnp.float32) + f1b[:]
    g = jax.nn.gelu(a, approximate=False)
    m = jnp.dot(g, f2w[:], preferred_element_type=jnp.float32) + f2b[:]
    x2_ref[0] = xb + ls2[:] * m


def _select_body(x2_ref, ca_ref, wuw, wub, nuw, nub, wtw, wtb, ntw, ntb,
                 gam, bet, out_ref):
    xb = x2_ref[0]
    cls = xb[0:1, :]
    m_cls = _ln(jnp.dot(cls, wuw[:], preferred_element_type=jnp.float32)
                + wub[:], nuw[:], nub[:])
    tri = _ln(jnp.dot(xb, wtw[:], preferred_element_type=jnp.float32)
              + wtb[:], ntw[:], ntb[:])
    prod = m_cls * tri
    inter_c = jnp.sqrt(jnp.sum(prod * prod, axis=-1, keepdims=True))
    ca_c = jnp.transpose(ca_ref[0], (1, 0))
    row_i = jax.lax.broadcasted_iota(jnp.int32, (_N, 1), 0)
    valid_c = row_i >= 1
    simp_c = ca_c * (1.0 + inter_c)
    vf = valid_c.astype(jnp.float32)
    mu = jnp.sum(simp_c * vf) / float(_P)
    dev = (simp_c - mu) * vf
    sd = jnp.sqrt(jnp.sum(dev * dev) / float(_P - 1))
    logit_c = gam[0, 0] * (simp_c - mu) / (sd + 1e-9) + bet[0, 0]
    logit_c = jnp.where(valid_c, logit_c, -1e30)
    logit_r = jnp.transpose(logit_c, (1, 0))
    lane_j = jax.lax.broadcasted_iota(jnp.int32, (_N, _N), 1)
    sub_i = jax.lax.broadcasted_iota(jnp.int32, (_N, _N), 0)
    lc = jnp.broadcast_to(logit_c, (_N, _N))
    lr = jnp.broadcast_to(logit_r, (_N, _N))
    beats = (lr > lc) | ((lr == lc) & (lane_j < sub_i))
    beats = beats & (lane_j >= 1)
    rank_c = jnp.sum(beats.astype(jnp.float32), axis=-1, keepdims=True)
    rank_r = jnp.transpose(rank_c, (1, 0))
    t = logit_r / _TAU
    tm = jnp.max(t, axis=-1, keepdims=True)
    te = jnp.exp(t - tm)
    y_soft = te / jnp.sum(te, axis=-1, keepdims=True)
    slot = jax.lax.broadcasted_iota(jnp.int32, (_K + 1, 1), 0).astype(jnp.float32)
    lane = jax.lax.broadcasted_iota(jnp.int32, (_K + 1, _N), 1)
    rk = jnp.broadcast_to(rank_r, (_K + 1, _N))
    oh = ((rk == slot - 1.0) & (lane >= 1)) | ((slot == 0.0) & (lane == 0))
    ohf = oh.astype(jnp.float32)
    toks = jnp.dot(ohf, xb, preferred_element_type=jnp.float32)
    p_sel = jax.lax.dot_general(ohf, y_soft, (((1,), (1,)), ((), ())),
                                preferred_element_type=jnp.float32)
    st = (1.0 - p_sel) + p_sel
    st = jnp.where(slot == 0.0, 1.0, st)
    out_ref[0] = toks * st


def _full(shape):
    nd = len(shape)
    return pl.BlockSpec(shape, lambda b: (0,) * nd)


def kernel(x, params):
    p = params
    r2 = lambda a: a.reshape(1, -1)
    f32 = jnp.float32

    attn = pl.pallas_call(
        _attn_body,
        grid=(_B,),
        in_specs=[
            pl.BlockSpec((1, _N, _D), lambda b: (b, 0, 0)),
            _full((1, _D)), _full((1, _D)),
            _full((_D, 3 * _D)), _full((1, 3 * _D)),
            _full((_D, _D)), _full((1, _D)), _full((1, _D)),
        ],
        out_specs=[
            pl.BlockSpec((1, _N, _D), lambda b: (b, 0, 0)),
            pl.BlockSpec((1, 1, _N), lambda b: (b, 0, 0)),
        ],
        out_shape=[
            jax.ShapeDtypeStruct((_B, _N, _D), f32),
            jax.ShapeDtypeStruct((_B, 1, _N), f32),
        ],
    )
    x1, ca = attn(x, r2(p['norm1_w']), r2(p['norm1_b']), p['qkv_w'],
                  r2(p['qkv_b']), p['proj_w'], r2(p['proj_b']), r2(p['ls1']))

    mlp = pl.pallas_call(
        _mlp_body,
        grid=(_B,),
        in_specs=[
            pl.BlockSpec((1, _N, _D), lambda b: (b, 0, 0)),
            _full((1, _D)), _full((1, _D)),
            _full((_D, 4 * _D)), _full((1, 4 * _D)),
            _full((4 * _D, _D)), _full((1, _D)), _full((1, _D)),
        ],
        out_specs=pl.BlockSpec((1, _N, _D), lambda b: (b, 0, 0)),
        out_shape=jax.ShapeDtypeStruct((_B, _N, _D), f32),
    )
    x2 = mlp(x1, r2(p['norm2_w']), r2(p['norm2_b']), p['fc1_w'],
             r2(p['fc1_b']), p['fc2_w'], r2(p['fc2_b']), r2(p['ls2']))

    sel = pl.pallas_call(
        _select_body,
        grid=(_B,),
        in_specs=[
            pl.BlockSpec((1, _N, _D), lambda b: (b, 0, 0)),
            pl.BlockSpec((1, 1, _N), lambda b: (b, 0, 0)),
            _full((_D, _D)), _full((1, _D)), _full((1, _D)), _full((1, _D)),
            _full((_D, _D)), _full((1, _D)), _full((1, _D)), _full((1, _D)),
            _full((1, 1)), _full((1, 1)),
        ],
        out_specs=pl.BlockSpec((1, _K + 1, _D), lambda b: (b, 0, 0)),
        out_shape=jax.ShapeDtypeStruct((_B, _K + 1, _D), f32),
    )
    toks = sel(x2, ca, p['wu_w'], r2(p['wu_b']), r2(p['normu_w']),
               r2(p['normu_b']), p['wtri_w'], r2(p['wtri_b']),
               r2(p['normtri_w']), r2(p['normtri_b']),
               p['gamma'].reshape(1, 1), p['beta'].reshape(1, 1))
    return toks


# speedup vs baseline: 2.6481x; 2.6481x over previous
import functools

import jax
import jax.numpy as jnp
from jax.experimental import pallas as pl
from jax.experimental.pallas import tpu as pltpu

_B, _N, _D, _H = 32, 577, 768, 12
_DH = _D // _H
_P = _N - 1
_K = 144
_TAU = 2.0


def _ln(v, w, b, eps=1e-5):
    mu = v.mean(axis=-1, keepdims=True)
    var = ((v - mu) ** 2).mean(axis=-1, keepdims=True)
    return (v - mu) / jnp.sqrt(var + eps) * w + b


def _attn_body(x_ref, mu_ref, var_ref, n1w, n1b, qkvw, qkvb, pw, pb, ls1,
               x1_ref, ca_ref):
    xb = x_ref[0]
    h = (xb - mu_ref[0]) / jnp.sqrt(var_ref[0] + 1e-5) * n1w[:] + n1b[:]
    qkv = jnp.dot(h, qkvw[:], preferred_element_type=jnp.float32) + qkvb[:]
    scale = _DH ** -0.5
    heads = []
    rows = []
    for hh in range(_H):
        q = qkv[:, hh * _DH:(hh + 1) * _DH]
        k = qkv[:, _D + hh * _DH:_D + (hh + 1) * _DH]
        v = qkv[:, 2 * _D + hh * _DH:2 * _D + (hh + 1) * _DH]
        s = jax.lax.dot_general(q, k, (((1,), (1,)), ((), ())),
                                preferred_element_type=jnp.float32) * scale
        m = jnp.max(s, axis=-1, keepdims=True)
        e = jnp.exp(s - m)
        p = e / jnp.sum(e, axis=-1, keepdims=True)
        heads.append(jnp.dot(p, v, preferred_element_type=jnp.float32))
        rows.append(p[0:1, :])
    o_cat = jnp.concatenate(heads, axis=1)
    out = jnp.dot(o_cat, pw[:], preferred_element_type=jnp.float32) + pb[:]
    x1_ref[0] = xb + ls1[:] * out
    ca_ref[0] = jnp.concatenate(rows, axis=0)


def _mlp_body(x1_ref, n2w, n2b, f1w, f1b, f2w, f2b, ls2, x2_ref):
    xb = x1_ref[0]
    h2 = _ln(xb, n2w[:], n2b[:])
    a = jnp.dot(h2, f1w[:], preferred_element_type=jnp.float32) + f1b[:]
    g = 0.5 * a * (1.0 + jax.lax.erf(a * 0.7071067811865476))
    m = jnp.dot(g, f2w[:], preferred_element_type=jnp.float32) + f2b[:]
    x2_ref[0] = xb + ls2[:] * m


def _select_body(x2_ref, ca_ref, wuw, wub, nuw, nub, wtw, wtb, ntw, ntb,
                 gam, bet, out_ref, lscr):
    xb = x2_ref[0]
    cls = xb[0:1, :]
    patches = xb[1:, :]
    m_cls = _ln(jnp.dot(cls, wuw[:], preferred_element_type=jnp.float32)
                + wub[:], nuw[:], nub[:])
    tri = _ln(jnp.dot(patches, wtw[:], preferred_element_type=jnp.float32)
              + wtb[:], ntw[:], ntb[:])
    prod = m_cls * tri
    inter_c = jnp.sqrt(jnp.sum(prod * prod, axis=-1, keepdims=True))
    inter_r = jnp.transpose(inter_c, (1, 0))
    base = ca_ref[0][:, 1:]
    simp = base * (1.0 + inter_r)
    mu = jnp.mean(simp, axis=-1, keepdims=True)
    ctr = simp - mu
    sd = jnp.sqrt(jnp.sum(ctr * ctr, axis=-1, keepdims=True) / float(_P - 1))
    logit = gam[0, 0] * (simp - mu) / (sd + 1e-9) + bet[0, 0]
    lscr[...] = logit
    logit_r = lscr[...]
    logit_c = jnp.transpose(logit_r, (1, 0))
    lane_j = jax.lax.broadcasted_iota(jnp.int32, (_P, _P), 1)
    sub_i = jax.lax.broadcasted_iota(jnp.int32, (_P, _P), 0)
    lc = jnp.broadcast_to(logit_c, (_P, _P))
    lr = jnp.broadcast_to(logit_r, (_P, _P))
    beats = (lr > lc) | ((lr == lc) & (lane_j < sub_i))
    rank_c = jnp.sum(beats.astype(jnp.float32), axis=-1, keepdims=True)
    rank_r = jnp.transpose(rank_c, (1, 0))
    t = logit_r / _TAU
    tm = jnp.max(t, axis=-1, keepdims=True)
    te = jnp.exp(t - tm)
    y_soft = te / jnp.sum(te, axis=-1, keepdims=True)
    slot = jax.lax.broadcasted_iota(jnp.int32, (_K, 1), 0).astype(jnp.float32)
    rk = jnp.broadcast_to(rank_r, (_K, _P))
    ohf = (rk == slot).astype(jnp.float32)
    toks = jnp.dot(ohf, patches, preferred_element_type=jnp.float32,
                   precision=jax.lax.Precision.HIGHEST)
    p_sel = jax.lax.dot_general(ohf, y_soft, (((1,), (1,)), ((), ())),
                                preferred_element_type=jnp.float32,
                                precision=jax.lax.Precision.HIGHEST)
    st = (1.0 - p_sel) + p_sel
    out_ref[0, 0:1, :] = cls
    out_ref[0, 1:, :] = toks * st


def _full(shape):
    nd = len(shape)
    return pl.BlockSpec(shape, lambda b: (0,) * nd)


def kernel(x, params):
    p = params
    r2 = lambda a: a.reshape(1, -1)
    f32 = jnp.float32

    attn = pl.pallas_call(
        _attn_body,
        grid=(_B,),
        in_specs=[
            pl.BlockSpec((1, _N, _D), lambda b: (b, 0, 0)),
            pl.BlockSpec((1, _N, 1), lambda b: (b, 0, 0)),
            pl.BlockSpec((1, _N, 1), lambda b: (b, 0, 0)),
            _full((1, _D)), _full((1, _D)),
            _full((_D, 3 * _D)), _full((1, 3 * _D)),
            _full((_D, _D)), _full((1, _D)), _full((1, _D)),
        ],
        out_specs=[
            pl.BlockSpec((1, _N, _D), lambda b: (b, 0, 0)),
            pl.BlockSpec((1, _H, _N), lambda b: (b, 0, 0)),
        ],
        out_shape=[
            jax.ShapeDtypeStruct((_B, _N, _D), f32),
            jax.ShapeDtypeStruct((_B, _H, _N), f32),
        ],
    )
    mu1 = x.mean(axis=-1, keepdims=True)
    var1 = x.var(axis=-1, keepdims=True)
    x1, ca_h = attn(x, mu1, var1, r2(p['norm1_w']), r2(p['norm1_b']), p['qkv_w'],
                    r2(p['qkv_b']), p['proj_w'], r2(p['proj_b']), r2(p['ls1']))
    ca = ca_h.mean(axis=1).reshape(_B, 1, _N)

    mlp = pl.pallas_call(
        _mlp_body,
        grid=(_B,),
        in_specs=[
            pl.BlockSpec((1, _N, _D), lambda b: (b, 0, 0)),
            _full((1, _D)), _full((1, _D)),
            _full((_D, 4 * _D)), _full((1, 4 * _D)),
            _full((4 * _D, _D)), _full((1, _D)), _full((1, _D)),
        ],
        out_specs=pl.BlockSpec((1, _N, _D), lambda b: (b, 0, 0)),
        out_shape=jax.ShapeDtypeStruct((_B, _N, _D), f32),
    )
    x2 = mlp(x1, r2(p['norm2_w']), r2(p['norm2_b']), p['fc1_w'],
             r2(p['fc1_b']), p['fc2_w'], r2(p['fc2_b']), r2(p['ls2']))

    sel = pl.pallas_call(
        _select_body,
        grid=(_B,),
        in_specs=[
            pl.BlockSpec((1, _N, _D), lambda b: (b, 0, 0)),
            pl.BlockSpec((1, 1, _N), lambda b: (b, 0, 0)),
            _full((_D, _D)), _full((1, _D)), _full((1, _D)), _full((1, _D)),
            _full((_D, _D)), _full((1, _D)), _full((1, _D)), _full((1, _D)),
            _full((1, 1)), _full((1, 1)),
        ],
        out_specs=pl.BlockSpec((1, _K + 1, _D), lambda b: (b, 0, 0)),
        out_shape=jax.ShapeDtypeStruct((_B, _K + 1, _D), f32),
        scratch_shapes=[pltpu.VMEM((1, _P), f32)],
    )
    toks = sel(x2, ca, p['wu_w'], r2(p['wu_b']), r2(p['normu_w']),
               r2(p['normu_b']), p['wtri_w'], r2(p['wtri_b']),
               r2(p['normtri_w']), r2(p['normtri_b']),
               p['gamma'].reshape(1, 1), p['beta'].reshape(1, 1))
    return toks
